# Initial kernel scaffold; baseline (speedup 1.0000x reference)
import jax
import jax.numpy as jnp
from jax.experimental import pallas as pl


def kernel(feats, edge_index, W1, b1, W2, b2):
    raise NotImplementedError("write your pallas kernel here")


# speedup vs baseline: 3.5091x; 3.5091x over previous
import jax
import jax.numpy as jnp
from jax import lax
from jax.experimental import pallas as pl
from jax.experimental.pallas import tpu as pltpu
from jax.experimental.pallas import tpu_sc as plsc

N_NODES = 10000
N_EDGES = 320000
D = 128

NC = 2
NS = 16
NW = NC * NS
L = 16

CH = 128
K = 80
R = 2
K2 = K // R
E_PAD = NW * K * CH
NPAD = 10240
RPT = NPAD // NS

_MESH = plsc.VectorSubcoreMesh(core_axis_name="c", subcore_axis_name="s")


def _zero_f32(ref, nrows, ncols):
    @pl.loop(0, nrows)
    def _(i):
        @pl.loop(0, ncols, step=L)
        def _(j):
            ref[i, pl.ds(j, L)] = jnp.zeros((L,), jnp.float32)


def _seg_sum_body(src_hbm, dst_hbm, table_hbm, out_hbm, *rest):
    (srcv, dstv, buf_a, buf_b, acc, sem_a, sem_b) = rest
    c = lax.axis_index("c")
    s = lax.axis_index("s")
    w = c * NS + s
    base = s * RPT

    _zero_f32(buf_a, CH, D)

    @pl.loop(0, RPT, step=CH)
    def _(r):
        pltpu.sync_copy(buf_a.at[pl.ds(0, CH)], acc.at[pl.ds(base + r, CH)])

    plsc.subcore_barrier()

    for r in range(R):
        pltpu.sync_copy(src_hbm.at[w, pl.ds(r * K2, K2)], srcv)
        pltpu.sync_copy(dst_hbm.at[w, pl.ds(r * K2, K2)], dstv)

        pltpu.async_copy(table_hbm.at[srcv.at[0]], buf_a, sem_a)
        pltpu.async_copy(table_hbm.at[srcv.at[1]], buf_b, sem_b)

        @pl.loop(0, K2, step=2)
        def _(j):
            pltpu.make_async_copy(table_hbm.at[pl.ds(0, CH)], buf_a, sem_a).wait()
            pltpu.sync_copy(buf_a, acc.at[dstv.at[j]], add=True)
            jn = jnp.where(j + 2 < K2, j + 2, 0)
            pltpu.async_copy(table_hbm.at[srcv.at[jn]], buf_a, sem_a)

            pltpu.make_async_copy(table_hbm.at[pl.ds(0, CH)], buf_b, sem_b).wait()
            pltpu.sync_copy(buf_b, acc.at[dstv.at[j + 1]], add=True)
            jm = jnp.where(j + 3 < K2, j + 3, 0)
            pltpu.async_copy(table_hbm.at[srcv.at[jm]], buf_b, sem_b)

        pltpu.make_async_copy(table_hbm.at[pl.ds(0, CH)], buf_a, sem_a).wait()
        pltpu.make_async_copy(table_hbm.at[pl.ds(0, CH)], buf_b, sem_b).wait()

    plsc.subcore_barrier()

    @pl.loop(0, RPT, step=CH)
    def _(r):
        pltpu.sync_copy(acc.at[pl.ds(base + r, CH)],
                        out_hbm.at[c, pl.ds(base + r, CH)])


_seg_sum = pl.kernel(
    _seg_sum_body,
    mesh=_MESH,
    out_type=jax.ShapeDtypeStruct((NC, NPAD, D), jnp.float32),
    scratch_types=[
        pltpu.VMEM((K2, CH), jnp.int32),
        pltpu.VMEM((K2, CH), jnp.int32),
        pltpu.VMEM((CH, D), jnp.float32),
        pltpu.VMEM((CH, D), jnp.float32),
        pltpu.VMEM_SHARED((NPAD, D), jnp.float32),
        pltpu.SemaphoreType.DMA,
        pltpu.SemaphoreType.DMA,
    ],
)


def _deg_body(dst_hbm, deg_hbm, dstv, onesv, dacc):
    c = lax.axis_index("c")
    s = lax.axis_index("s")
    w = c * NS + s
    base = s * RPT

    pltpu.sync_copy(dst_hbm.at[w], dstv)

    _zero_f32(onesv, CH, D)

    @pl.loop(0, RPT, step=CH)
    def _(r):
        pltpu.sync_copy(onesv.at[pl.ds(0, CH)], dacc.at[pl.ds(base + r, CH)])

    @pl.loop(0, CH)
    def _(i):
        @pl.loop(0, D, step=L)
        def _(jj):
            onesv[i, pl.ds(jj, L)] = jnp.ones((L,), jnp.float32)

    plsc.subcore_barrier()

    @pl.loop(0, K)
    def _(j):
        pltpu.sync_copy(onesv, dacc.at[dstv.at[j]], add=True)

    plsc.subcore_barrier()

    @pl.loop(0, RPT, step=CH)
    def _(r):
        pltpu.sync_copy(dacc.at[pl.ds(base + r, CH)],
                        deg_hbm.at[c, pl.ds(base + r, CH)])


_deg_sum = pl.kernel(
    _deg_body,
    mesh=_MESH,
    out_type=jax.ShapeDtypeStruct((NC, NPAD, D), jnp.float32),
    scratch_types=[
        pltpu.VMEM((K, CH), jnp.int32),
        pltpu.VMEM((CH, D), jnp.float32),
        pltpu.VMEM_SHARED((NPAD, D), jnp.float32),
    ],
)


BM = 512


def _tc_body1(p0, p1, h, d0, d1, wt, b, o1, o2):
    inv = 1.0 / (d0[:, 0:1] + d1[:, 0:1] + 1.0)
    x = (p0[...] + p1[...] + h[...]) * inv
    y = jnp.dot(x, wt[...], preferred_element_type=jnp.float32) + b[...]
    o1[...] = y
    o2[...] = jnp.maximum(y, 0.0)


def _tc_body2(p0, p1, h, d0, d1, wt, b, o1):
    inv = 1.0 / (d0[:, 0:1] + d1[:, 0:1] + 1.0)
    x = (p0[...] + p1[...] + h[...]) * inv
    o1[...] = jnp.dot(x, wt[...], preferred_element_type=jnp.float32) + b[...]


def _tc_layer(p, dp, h, wt, b, relu):
    grid = (NPAD // BM,)
    row = pl.BlockSpec((BM, D), lambda i: (i, 0))
    deg = pl.BlockSpec((BM, D), lambda i: (i, 0))
    full = pl.BlockSpec((D, D), lambda i: (0, 0))
    bias = pl.BlockSpec((1, D), lambda i: (0, 0))
    out = jax.ShapeDtypeStruct((NPAD, D), jnp.float32)
    if relu:
        return pl.pallas_call(
            _tc_body1,
            grid=grid,
            in_specs=[row, row, row, deg, deg, full, bias],
            out_specs=[row, row],
            out_shape=[out, out],
        )(p[0], p[1], h, dp[0], dp[1], wt, b)
    return pl.pallas_call(
        _tc_body2,
        grid=grid,
        in_specs=[row, row, row, deg, deg, full, bias],
        out_specs=row,
        out_shape=out,
    )(p[0], p[1], h, dp[0], dp[1], wt, b)


def kernel(feats, edge_index, W1, b1, W2, b2):
    src = edge_index[0].astype(jnp.int32)
    dst = edge_index[1].astype(jnp.int32)
    pad = E_PAD - N_EDGES
    src3 = jnp.concatenate([src, jnp.zeros((pad,), jnp.int32)]).reshape(NW, K, CH)
    dst3 = jnp.concatenate(
        [dst, jnp.full((pad,), N_NODES, jnp.int32)]).reshape(NW, K, CH)
    hpad = jnp.pad(feats, ((0, NPAD - N_NODES), (0, 0)))
    w1t = W1.T
    w2t = W2.T
    b1r = b1.reshape(1, D)
    b2r = b2.reshape(1, D)

    dp = _deg_sum(dst3)
    p1 = _seg_sum(src3, dst3, hpad)
    h1, r = _tc_layer(p1, dp, hpad, w1t, b1r, True)
    p2 = _seg_sum(src3, dst3, r)
    h2 = _tc_layer(p2, dp, r, w2t, b2r, False)
    return (h1[:N_NODES], h2[:N_NODES])


# speedup vs baseline: 3.5114x; 1.0007x over previous
import jax
import jax.numpy as jnp
from jax import lax
from jax.experimental import pallas as pl
from jax.experimental.pallas import tpu as pltpu
from jax.experimental.pallas import tpu_sc as plsc

N_NODES = 10000
N_EDGES = 320000
D = 128

NC = 2
NS = 16
NW = NC * NS
L = 16

CH = 128
K = 80
R = 2
K2 = K // R
E_PAD = NW * K * CH
NPAD = 10240
RPT = NPAD // NS

_MESH = plsc.VectorSubcoreMesh(core_axis_name="c", subcore_axis_name="s")


def _zero_f32(ref, nrows, ncols):
    @pl.loop(0, nrows)
    def _(i):
        @pl.loop(0, ncols, step=L)
        def _(j):
            ref[i, pl.ds(j, L)] = jnp.zeros((L,), jnp.float32)


def _seg_sum_body(src_hbm, dst_hbm, table_hbm, out_hbm, *rest):
    (srcv, dstv, buf_a, buf_b, acc, sem_a, sem_b) = rest
    c = lax.axis_index("c")
    s = lax.axis_index("s")
    w = c * NS + s
    base = s * RPT

    _zero_f32(buf_a, CH, D)

    @pl.loop(0, RPT, step=CH)
    def _(r):
        pltpu.sync_copy(buf_a.at[pl.ds(0, CH)], acc.at[pl.ds(base + r, CH)])

    plsc.subcore_barrier()

    for r in range(R):
        pltpu.sync_copy(src_hbm.at[w, pl.ds(r * K2, K2)], srcv)
        pltpu.sync_copy(dst_hbm.at[w, pl.ds(r * K2, K2)], dstv)

        pltpu.async_copy(table_hbm.at[srcv.at[0]], buf_a, sem_a)
        pltpu.async_copy(table_hbm.at[srcv.at[1]], buf_b, sem_b)

        @pl.loop(0, K2, step=2)
        def _(j):
            pltpu.make_async_copy(table_hbm.at[pl.ds(0, CH)], buf_a, sem_a).wait()
            pltpu.sync_copy(buf_a, acc.at[dstv.at[j]], add=True)
            jn = jnp.where(j + 2 < K2, j + 2, 0)
            pltpu.async_copy(table_hbm.at[srcv.at[jn]], buf_a, sem_a)

            pltpu.make_async_copy(table_hbm.at[pl.ds(0, CH)], buf_b, sem_b).wait()
            pltpu.sync_copy(buf_b, acc.at[dstv.at[j + 1]], add=True)
            jm = jnp.where(j + 3 < K2, j + 3, 0)
            pltpu.async_copy(table_hbm.at[srcv.at[jm]], buf_b, sem_b)

        pltpu.make_async_copy(table_hbm.at[pl.ds(0, CH)], buf_a, sem_a).wait()
        pltpu.make_async_copy(table_hbm.at[pl.ds(0, CH)], buf_b, sem_b).wait()

    plsc.subcore_barrier()

    @pl.loop(0, RPT, step=CH)
    def _(r):
        pltpu.sync_copy(acc.at[pl.ds(base + r, CH)],
                        out_hbm.at[c, pl.ds(base + r, CH)])


_seg_sum = pl.kernel(
    _seg_sum_body,
    mesh=_MESH,
    out_type=jax.ShapeDtypeStruct((NC, NPAD, D), jnp.float32),
    scratch_types=[
        pltpu.VMEM((K2, CH), jnp.int32),
        pltpu.VMEM((K2, CH), jnp.int32),
        pltpu.VMEM((CH, D), jnp.float32),
        pltpu.VMEM((CH, D), jnp.float32),
        pltpu.VMEM_SHARED((NPAD, D), jnp.float32),
        pltpu.SemaphoreType.DMA,
        pltpu.SemaphoreType.DMA,
    ],
)


def _deg_body(dst_hbm, deg_hbm, dstv, onesv, dacc):
    c = lax.axis_index("c")
    s = lax.axis_index("s")
    w = c * NS + s
    base = s * RPT

    pltpu.sync_copy(dst_hbm.at[w], dstv)

    _zero_f32(onesv, CH, D)

    @pl.loop(0, RPT, step=CH)
    def _(r):
        pltpu.sync_copy(onesv.at[pl.ds(0, CH)], dacc.at[pl.ds(base + r, CH)])

    @pl.loop(0, CH)
    def _(i):
        @pl.loop(0, D, step=L)
        def _(jj):
            onesv[i, pl.ds(jj, L)] = jnp.ones((L,), jnp.float32)

    plsc.subcore_barrier()

    @pl.loop(0, K)
    def _(j):
        pltpu.sync_copy(onesv, dacc.at[dstv.at[j]], add=True)

    plsc.subcore_barrier()

    @pl.loop(0, RPT, step=CH)
    def _(r):
        pltpu.sync_copy(dacc.at[pl.ds(base + r, CH)],
                        deg_hbm.at[c, pl.ds(base + r, CH)])


_deg_sum = pl.kernel(
    _deg_body,
    mesh=_MESH,
    out_type=jax.ShapeDtypeStruct((NC, NPAD, D), jnp.float32),
    scratch_types=[
        pltpu.VMEM((K, CH), jnp.int32),
        pltpu.VMEM((CH, D), jnp.float32),
        pltpu.VMEM_SHARED((NPAD, D), jnp.float32),
    ],
)


BM = 512


def _tc_body1(p0, p1, h, d0, d1, wt, b, o1, o2):
    inv = 1.0 / (d0[:, 0:1] + d1[:, 0:1] + 1.0)
    x = (p0[...] + p1[...] + h[...]) * inv
    y = jnp.dot(x, wt[...], preferred_element_type=jnp.float32) + b[...]
    o1[...] = y
    o2[...] = jnp.maximum(y, 0.0)


def _tc_body2(p0, p1, h, d0, d1, wt, b, o1):
    inv = 1.0 / (d0[:, 0:1] + d1[:, 0:1] + 1.0)
    x = (p0[...] + p1[...] + h[...]) * inv
    o1[...] = jnp.dot(x, wt[...], preferred_element_type=jnp.float32) + b[...]


def _tc_layer(p, dp, h, wt, b, relu):
    grid = (NPAD // BM,)
    row = pl.BlockSpec((BM, D), lambda i: (i, 0))
    deg = pl.BlockSpec((BM, D), lambda i: (i, 0))
    full = pl.BlockSpec((D, D), lambda i: (0, 0))
    bias = pl.BlockSpec((1, D), lambda i: (0, 0))
    out = jax.ShapeDtypeStruct((NPAD, D), jnp.float32)
    if relu:
        return pl.pallas_call(
            _tc_body1,
            grid=grid,
            in_specs=[row, row, row, deg, deg, full, bias],
            out_specs=[row, row],
            out_shape=[out, out],
        )(p[0], p[1], h, dp[0], dp[1], wt, b)
    return pl.pallas_call(
        _tc_body2,
        grid=grid,
        in_specs=[row, row, row, deg, deg, full, bias],
        out_specs=row,
        out_shape=out,
    )(p[0], p[1], h, dp[0], dp[1], wt, b)


def kernel(feats, edge_index, W1, b1, W2, b2):
    src = edge_index[0].astype(jnp.int32)
    dst = edge_index[1].astype(jnp.int32)
    pad = E_PAD - N_EDGES
    src3 = jnp.concatenate([src, jnp.zeros((pad,), jnp.int32)]).reshape(NW, K, CH)
    pad_dst = N_NODES + (jnp.arange(pad, dtype=jnp.int32) % (NPAD - N_NODES))
    dst3 = jnp.concatenate([dst, pad_dst]).reshape(NW, K, CH)
    hpad = jnp.pad(feats, ((0, NPAD - N_NODES), (0, 0)))
    w1t = W1.T
    w2t = W2.T
    b1r = b1.reshape(1, D)
    b2r = b2.reshape(1, D)

    dp = _deg_sum(dst3)
    p1 = _seg_sum(src3, dst3, hpad)
    h1, r = _tc_layer(p1, dp, hpad, w1t, b1r, True)
    p2 = _seg_sum(src3, dst3, r)
    h2 = _tc_layer(p2, dp, r, w2t, b2r, False)
    return (h1[:N_NODES], h2[:N_NODES])


# speedup vs baseline: 3.5156x; 1.0012x over previous
import jax
import jax.numpy as jnp
from jax import lax
from jax.experimental import pallas as pl
from jax.experimental.pallas import tpu as pltpu
from jax.experimental.pallas import tpu_sc as plsc

N_NODES = 10000
N_EDGES = 320000
D = 128

NC = 2
NS = 16
NW = NC * NS
L = 16

CH = 128
K = 80
R = 2
K2 = K // R
E_PAD = NW * K * CH
NPAD = 10240
RPT = NPAD // NS

_MESH = plsc.VectorSubcoreMesh(core_axis_name="c", subcore_axis_name="s")


def _zero_f32(ref, nrows, ncols):
    @pl.loop(0, nrows)
    def _(i):
        @pl.loop(0, ncols, step=L)
        def _(j):
            ref[i, pl.ds(j, L)] = jnp.zeros((L,), jnp.float32)


def _seg_sum_body(src_hbm, dst_hbm, table_hbm, out_hbm, *rest):
    (srcv, dstv, buf_a, buf_b, acc, sem_a, sem_b) = rest
    c = lax.axis_index("c")
    s = lax.axis_index("s")
    w = s * NC + c
    base = s * RPT

    _zero_f32(buf_a, CH, D)

    @pl.loop(0, RPT, step=CH)
    def _(r):
        pltpu.sync_copy(buf_a.at[pl.ds(0, CH)], acc.at[pl.ds(base + r, CH)])

    plsc.subcore_barrier()

    for r in range(R):
        pltpu.sync_copy(src_hbm.at[w, pl.ds(r * K2, K2)], srcv)
        pltpu.sync_copy(dst_hbm.at[w, pl.ds(r * K2, K2)], dstv)

        pltpu.async_copy(table_hbm.at[srcv.at[0]], buf_a, sem_a)
        pltpu.async_copy(table_hbm.at[srcv.at[1]], buf_b, sem_b)

        @pl.loop(0, K2, step=2)
        def _(j):
            pltpu.make_async_copy(table_hbm.at[pl.ds(0, CH)], buf_a, sem_a).wait()
            pltpu.sync_copy(buf_a, acc.at[dstv.at[j]], add=True)
            jn = jnp.where(j + 2 < K2, j + 2, 0)
            pltpu.async_copy(table_hbm.at[srcv.at[jn]], buf_a, sem_a)

            pltpu.make_async_copy(table_hbm.at[pl.ds(0, CH)], buf_b, sem_b).wait()
            pltpu.sync_copy(buf_b, acc.at[dstv.at[j + 1]], add=True)
            jm = jnp.where(j + 3 < K2, j + 3, 0)
            pltpu.async_copy(table_hbm.at[srcv.at[jm]], buf_b, sem_b)

        pltpu.make_async_copy(table_hbm.at[pl.ds(0, CH)], buf_a, sem_a).wait()
        pltpu.make_async_copy(table_hbm.at[pl.ds(0, CH)], buf_b, sem_b).wait()

    plsc.subcore_barrier()

    @pl.loop(0, RPT, step=CH)
    def _(r):
        pltpu.sync_copy(acc.at[pl.ds(base + r, CH)],
                        out_hbm.at[c, pl.ds(base + r, CH)])


_seg_sum = pl.kernel(
    _seg_sum_body,
    mesh=_MESH,
    out_type=jax.ShapeDtypeStruct((NC, NPAD, D), jnp.float32),
    scratch_types=[
        pltpu.VMEM((K2, CH), jnp.int32),
        pltpu.VMEM((K2, CH), jnp.int32),
        pltpu.VMEM((CH, D), jnp.float32),
        pltpu.VMEM((CH, D), jnp.float32),
        pltpu.VMEM_SHARED((NPAD, D), jnp.float32),
        pltpu.SemaphoreType.DMA,
        pltpu.SemaphoreType.DMA,
    ],
)


def _deg_body(dst_hbm, deg_hbm, dstv, onesv, dacc):
    c = lax.axis_index("c")
    s = lax.axis_index("s")
    w = c * NS + s
    base = s * RPT

    pltpu.sync_copy(dst_hbm.at[w], dstv)

    _zero_f32(onesv, CH, D)

    @pl.loop(0, RPT, step=CH)
    def _(r):
        pltpu.sync_copy(onesv.at[pl.ds(0, CH)], dacc.at[pl.ds(base + r, CH)])

    @pl.loop(0, CH)
    def _(i):
        @pl.loop(0, D, step=L)
        def _(jj):
            onesv[i, pl.ds(jj, L)] = jnp.ones((L,), jnp.float32)

    plsc.subcore_barrier()

    @pl.loop(0, K)
    def _(j):
        pltpu.sync_copy(onesv, dacc.at[dstv.at[j]], add=True)

    plsc.subcore_barrier()

    @pl.loop(0, RPT, step=CH)
    def _(r):
        pltpu.sync_copy(dacc.at[pl.ds(base + r, CH)],
                        deg_hbm.at[c, pl.ds(base + r, CH)])


_deg_sum = pl.kernel(
    _deg_body,
    mesh=_MESH,
    out_type=jax.ShapeDtypeStruct((NC, NPAD, D), jnp.float32),
    scratch_types=[
        pltpu.VMEM((K, CH), jnp.int32),
        pltpu.VMEM((CH, D), jnp.float32),
        pltpu.VMEM_SHARED((NPAD, D), jnp.float32),
    ],
)


BM = 512


def _tc_body1(p0, p1, h, d0, d1, wt, b, o1, o2):
    inv = 1.0 / (d0[:, 0:1] + d1[:, 0:1] + 1.0)
    x = (p0[...] + p1[...] + h[...]) * inv
    y = jnp.dot(x, wt[...], preferred_element_type=jnp.float32) + b[...]
    o1[...] = y
    o2[...] = jnp.maximum(y, 0.0)


def _tc_body2(p0, p1, h, d0, d1, wt, b, o1):
    inv = 1.0 / (d0[:, 0:1] + d1[:, 0:1] + 1.0)
    x = (p0[...] + p1[...] + h[...]) * inv
    o1[...] = jnp.dot(x, wt[...], preferred_element_type=jnp.float32) + b[...]


def _tc_layer(p, dp, h, wt, b, relu):
    grid = (NPAD // BM,)
    row = pl.BlockSpec((BM, D), lambda i: (i, 0))
    deg = pl.BlockSpec((BM, D), lambda i: (i, 0))
    full = pl.BlockSpec((D, D), lambda i: (0, 0))
    bias = pl.BlockSpec((1, D), lambda i: (0, 0))
    out = jax.ShapeDtypeStruct((NPAD, D), jnp.float32)
    if relu:
        return pl.pallas_call(
            _tc_body1,
            grid=grid,
            in_specs=[row, row, row, deg, deg, full, bias],
            out_specs=[row, row],
            out_shape=[out, out],
        )(p[0], p[1], h, dp[0], dp[1], wt, b)
    return pl.pallas_call(
        _tc_body2,
        grid=grid,
        in_specs=[row, row, row, deg, deg, full, bias],
        out_specs=row,
        out_shape=out,
    )(p[0], p[1], h, dp[0], dp[1], wt, b)


def kernel(feats, edge_index, W1, b1, W2, b2):
    src = edge_index[0].astype(jnp.int32)
    dst = edge_index[1].astype(jnp.int32)
    pad = E_PAD - N_EDGES
    src3 = jnp.concatenate([src, jnp.zeros((pad,), jnp.int32)]).reshape(NW, K, CH)
    pad_dst = N_NODES + (jnp.arange(pad, dtype=jnp.int32) % (NPAD - N_NODES))
    dst3 = jnp.concatenate([dst, pad_dst]).reshape(NW, K, CH)
    hpad = jnp.pad(feats, ((0, NPAD - N_NODES), (0, 0)))
    w1t = W1.T
    w2t = W2.T
    b1r = b1.reshape(1, D)
    b2r = b2.reshape(1, D)

    dp = _deg_sum(dst3)
    p1 = _seg_sum(src3, dst3, hpad)
    h1, r = _tc_layer(p1, dp, hpad, w1t, b1r, True)
    p2 = _seg_sum(src3, dst3, r)
    h2 = _tc_layer(p2, dp, r, w2t, b2r, False)
    return (h1[:N_NODES], h2[:N_NODES])


# speedup vs baseline: 9.7969x; 2.7867x over previous
import jax
import jax.numpy as jnp
from jax import lax
from jax.experimental import pallas as pl
from jax.experimental.pallas import tpu as pltpu
from jax.experimental.pallas import tpu_sc as plsc

N_NODES = 10000
N_EDGES = 320000
D = 128

NC = 2
NS = 16
NW = NC * NS
L = 16

CH = 128
K = 80
R = 2
K2 = K // R
E_PAD = NW * K * CH
NPAD = 10240
RPT = NPAD // NS

_MESH = plsc.VectorSubcoreMesh(core_axis_name="c", subcore_axis_name="s")


def _zero_f32(ref, nrows, ncols):
    @pl.loop(0, nrows)
    def _(i):
        @pl.loop(0, ncols, step=L)
        def _(j):
            ref[i, pl.ds(j, L)] = jnp.zeros((L,), jnp.float32)


def _seg_sum_body(src_hbm, dst_hbm, table_hbm, out_hbm, *rest):
    (srcv, dstv, buf_a, buf_b, acc, sem_a, sem_b) = rest
    c = lax.axis_index("c")
    s = lax.axis_index("s")
    w = s * NC + c
    base = s * RPT

    _zero_f32(buf_a, CH, D)

    @pl.loop(0, RPT, step=CH)
    def _(r):
        pltpu.sync_copy(buf_a.at[pl.ds(0, CH)], acc.at[pl.ds(base + r, CH)])

    plsc.subcore_barrier()

    for r in range(R):
        pltpu.sync_copy(src_hbm.at[w, pl.ds(r * K2, K2)], srcv)
        pltpu.sync_copy(dst_hbm.at[w, pl.ds(r * K2, K2)], dstv)

        pltpu.async_copy(table_hbm.at[srcv.at[0]], buf_a, sem_a)
        pltpu.async_copy(table_hbm.at[srcv.at[1]], buf_b, sem_b)

        @pl.loop(0, K2, step=2)
        def _(j):
            pltpu.make_async_copy(table_hbm.at[pl.ds(0, CH)], buf_a, sem_a).wait()
            pltpu.sync_copy(buf_a, acc.at[dstv.at[j]], add=True)
            jn = jnp.where(j + 2 < K2, j + 2, 0)
            pltpu.async_copy(table_hbm.at[srcv.at[jn]], buf_a, sem_a)

            pltpu.make_async_copy(table_hbm.at[pl.ds(0, CH)], buf_b, sem_b).wait()
            pltpu.sync_copy(buf_b, acc.at[dstv.at[j + 1]], add=True)
            jm = jnp.where(j + 3 < K2, j + 3, 0)
            pltpu.async_copy(table_hbm.at[srcv.at[jm]], buf_b, sem_b)

        pltpu.make_async_copy(table_hbm.at[pl.ds(0, CH)], buf_a, sem_a).wait()
        pltpu.make_async_copy(table_hbm.at[pl.ds(0, CH)], buf_b, sem_b).wait()

    plsc.subcore_barrier()

    @pl.loop(0, RPT, step=CH)
    def _(r):
        pltpu.sync_copy(acc.at[pl.ds(base + r, CH)],
                        out_hbm.at[c, pl.ds(base + r, CH)])


_seg_sum = pl.kernel(
    _seg_sum_body,
    mesh=_MESH,
    out_type=jax.ShapeDtypeStruct((NC, NPAD, D), jnp.float32),
    scratch_types=[
        pltpu.VMEM((K2, CH), jnp.int32),
        pltpu.VMEM((K2, CH), jnp.int32),
        pltpu.VMEM((CH, D), jnp.float32),
        pltpu.VMEM((CH, D), jnp.float32),
        pltpu.VMEM_SHARED((NPAD, D), jnp.float32),
        pltpu.SemaphoreType.DMA,
        pltpu.SemaphoreType.DMA,
    ],
)


def _deg_body(dst_hbm, deg_hbm, dstv, onesv, dacc):
    c = lax.axis_index("c")
    s = lax.axis_index("s")
    w = c * NS + s
    base = s * RPT

    pltpu.sync_copy(dst_hbm.at[w], dstv)

    _zero_f32(onesv, CH, D)

    @pl.loop(0, RPT, step=CH)
    def _(r):
        pltpu.sync_copy(onesv.at[pl.ds(0, CH)], dacc.at[pl.ds(base + r, CH)])

    @pl.loop(0, CH)
    def _(i):
        @pl.loop(0, D, step=L)
        def _(jj):
            onesv[i, pl.ds(jj, L)] = jnp.ones((L,), jnp.float32)

    plsc.subcore_barrier()

    @pl.loop(0, K)
    def _(j):
        pltpu.sync_copy(onesv, dacc.at[dstv.at[j]], add=True)

    plsc.subcore_barrier()

    @pl.loop(0, RPT, step=CH)
    def _(r):
        pltpu.sync_copy(dacc.at[pl.ds(base + r, CH)],
                        deg_hbm.at[c, pl.ds(base + r, CH)])


_deg_sum = pl.kernel(
    _deg_body,
    mesh=_MESH,
    out_type=jax.ShapeDtypeStruct((NC, NPAD, D), jnp.float32),
    scratch_types=[
        pltpu.VMEM((K, CH), jnp.int32),
        pltpu.VMEM((CH, D), jnp.float32),
        pltpu.VMEM_SHARED((NPAD, D), jnp.float32),
    ],
)


BM = 512


def _tc_body1(p0, p1, h, d0, d1, wt, b, o1, o2):
    inv = 1.0 / (d0[:, 0:1] + d1[:, 0:1] + 1.0)
    x = (p0[...] + p1[...] + h[...]) * inv
    y = jnp.dot(x, wt[...], preferred_element_type=jnp.float32) + b[...]
    o1[...] = y
    o2[...] = jnp.maximum(y, 0.0)


def _tc_body2(p0, p1, h, d0, d1, wt, b, o1):
    inv = 1.0 / (d0[:, 0:1] + d1[:, 0:1] + 1.0)
    x = (p0[...] + p1[...] + h[...]) * inv
    o1[...] = jnp.dot(x, wt[...], preferred_element_type=jnp.float32) + b[...]


def _tc_layer(p, dp, h, wt, b, relu):
    grid = (NPAD // BM,)
    row = pl.BlockSpec((BM, D), lambda i: (i, 0))
    deg = pl.BlockSpec((BM, D), lambda i: (i, 0))
    full = pl.BlockSpec((D, D), lambda i: (0, 0))
    bias = pl.BlockSpec((1, D), lambda i: (0, 0))
    out = jax.ShapeDtypeStruct((NPAD, D), jnp.float32)
    if relu:
        return pl.pallas_call(
            _tc_body1,
            grid=grid,
            in_specs=[row, row, row, deg, deg, full, bias],
            out_specs=[row, row],
            out_shape=[out, out],
        )(p[0], p[1], h, dp[0], dp[1], wt, b)
    return pl.pallas_call(
        _tc_body2,
        grid=grid,
        in_specs=[row, row, row, deg, deg, full, bias],
        out_specs=row,
        out_shape=out,
    )(p[0], p[1], h, dp[0], dp[1], wt, b)


def kernel(feats, edge_index, W1, b1, W2, b2):
    src = edge_index[0].astype(jnp.int32)
    dst = edge_index[1].astype(jnp.int32)
    pad = E_PAD - N_EDGES
    pad_ix = N_NODES + (jnp.arange(pad, dtype=jnp.int32) % (NPAD - N_NODES))
    src3 = jnp.concatenate([src, pad_ix]).reshape(NW, K, CH)
    dst3 = jnp.concatenate([dst, pad_ix]).reshape(NW, K, CH)
    hpad = jnp.pad(feats, ((0, NPAD - N_NODES), (0, 0)))
    w1t = W1.T
    w2t = W2.T
    b1r = b1.reshape(1, D)
    b2r = b2.reshape(1, D)

    dp = _deg_sum(dst3)
    p1 = _seg_sum(src3, dst3, hpad)
    h1, r = _tc_layer(p1, dp, hpad, w1t, b1r, True)
    p2 = _seg_sum(src3, dst3, r)
    h2 = _tc_layer(p2, dp, r, w2t, b2r, False)
    return (h1[:N_NODES], h2[:N_NODES])


# speedup vs baseline: 9.9681x; 1.0175x over previous
import jax
import jax.numpy as jnp
from jax import lax
from jax.experimental import pallas as pl
from jax.experimental.pallas import tpu as pltpu
from jax.experimental.pallas import tpu_sc as plsc

N_NODES = 10000
N_EDGES = 320000
D = 128

NC = 2
NS = 16
NW = NC * NS
L = 16

CH = 128
K = 80
R = 2
K2 = K // R
E_PAD = NW * K * CH
NPAD = 10240
RPT = NPAD // NS

_MESH = plsc.VectorSubcoreMesh(core_axis_name="c", subcore_axis_name="s")


def _zero_f32(ref, nrows, ncols):
    @pl.loop(0, nrows)
    def _(i):
        @pl.loop(0, ncols, step=L)
        def _(j):
            ref[i, pl.ds(j, L)] = jnp.zeros((L,), jnp.float32)


def _seg_sum_body(src_hbm, dst_hbm, table_hbm, out_hbm, *rest):
    (srcv, dstv, buf_a, buf_b, acc, sem_a, sem_b) = rest
    c = lax.axis_index("c")
    s = lax.axis_index("s")
    w = s * NC + c
    base = s * RPT

    _zero_f32(buf_a, CH, D)

    @pl.loop(0, RPT, step=CH)
    def _(r):
        pltpu.sync_copy(buf_a.at[pl.ds(0, CH)], acc.at[pl.ds(base + r, CH)])

    plsc.subcore_barrier()

    for r in range(R):
        pltpu.sync_copy(src_hbm.at[w, pl.ds(r * K2, K2)], srcv)
        pltpu.sync_copy(dst_hbm.at[w, pl.ds(r * K2, K2)], dstv)

        pltpu.async_copy(table_hbm.at[srcv.at[0]], buf_a, sem_a)
        pltpu.async_copy(table_hbm.at[srcv.at[1]], buf_b, sem_b)

        @pl.loop(0, K2, step=2)
        def _(j):
            pltpu.make_async_copy(table_hbm.at[pl.ds(0, CH)], buf_a, sem_a).wait()
            pltpu.sync_copy(buf_a, acc.at[dstv.at[j]], add=True)
            jn = jnp.where(j + 2 < K2, j + 2, 0)
            pltpu.async_copy(table_hbm.at[srcv.at[jn]], buf_a, sem_a)

            pltpu.make_async_copy(table_hbm.at[pl.ds(0, CH)], buf_b, sem_b).wait()
            pltpu.sync_copy(buf_b, acc.at[dstv.at[j + 1]], add=True)
            jm = jnp.where(j + 3 < K2, j + 3, 0)
            pltpu.async_copy(table_hbm.at[srcv.at[jm]], buf_b, sem_b)

        pltpu.make_async_copy(table_hbm.at[pl.ds(0, CH)], buf_a, sem_a).wait()
        pltpu.make_async_copy(table_hbm.at[pl.ds(0, CH)], buf_b, sem_b).wait()

    plsc.subcore_barrier()

    @pl.loop(0, RPT, step=CH)
    def _(r):
        pltpu.sync_copy(acc.at[pl.ds(base + r, CH)],
                        out_hbm.at[c, pl.ds(base + r, CH)])


_seg_sum = pl.kernel(
    _seg_sum_body,
    mesh=_MESH,
    out_type=jax.ShapeDtypeStruct((NC, NPAD, D), jnp.float32),
    scratch_types=[
        pltpu.VMEM((K2, CH), jnp.int32),
        pltpu.VMEM((K2, CH), jnp.int32),
        pltpu.VMEM((CH, D), jnp.float32),
        pltpu.VMEM((CH, D), jnp.float32),
        pltpu.VMEM_SHARED((NPAD, D), jnp.float32),
        pltpu.SemaphoreType.DMA,
        pltpu.SemaphoreType.DMA,
    ],
)


def _deg_body(dst_hbm, deg_hbm, dstv, onesv, dacc):
    c = lax.axis_index("c")
    s = lax.axis_index("s")
    w = c * NS + s
    base = s * RPT

    pltpu.sync_copy(dst_hbm.at[w], dstv)

    _zero_f32(onesv, CH, D)

    @pl.loop(0, RPT, step=CH)
    def _(r):
        pltpu.sync_copy(onesv.at[pl.ds(0, CH)], dacc.at[pl.ds(base + r, CH)])

    @pl.loop(0, CH)
    def _(i):
        @pl.loop(0, D, step=L)
        def _(jj):
            onesv[i, pl.ds(jj, L)] = jnp.ones((L,), jnp.float32)

    plsc.subcore_barrier()

    @pl.loop(0, K)
    def _(j):
        pltpu.sync_copy(onesv, dacc.at[dstv.at[j]], add=True)

    plsc.subcore_barrier()

    @pl.loop(0, RPT, step=CH)
    def _(r):
        pltpu.sync_copy(dacc.at[pl.ds(base + r, CH)],
                        deg_hbm.at[c, pl.ds(base + r, CH)])


_deg_sum = pl.kernel(
    _deg_body,
    mesh=_MESH,
    out_type=jax.ShapeDtypeStruct((NC, NPAD, D), jnp.float32),
    scratch_types=[
        pltpu.VMEM((K, CH), jnp.int32),
        pltpu.VMEM((CH, D), jnp.float32),
        pltpu.VMEM_SHARED((NPAD, D), jnp.float32),
    ],
)


BM = 400


def _tc_body1(p0, p1, h, d0, d1, wt, b, o1, o2):
    inv = 1.0 / (d0[:, 0:1] + d1[:, 0:1] + 1.0)
    x = (p0[...] + p1[...] + h[...]) * inv
    y = jnp.dot(x, wt[...], preferred_element_type=jnp.float32) + b[...]
    o1[...] = y
    o2[...] = jnp.maximum(y, 0.0)


def _tc_body2(p0, p1, h, d0, d1, wt, b, o1):
    inv = 1.0 / (d0[:, 0:1] + d1[:, 0:1] + 1.0)
    x = (p0[...] + p1[...] + h[...]) * inv
    o1[...] = jnp.dot(x, wt[...], preferred_element_type=jnp.float32) + b[...]


def _tc_layer(p, dp, h, wt, b, relu):
    grid = (N_NODES // BM,)
    row = pl.BlockSpec((BM, D), lambda i: (i, 0))
    deg = pl.BlockSpec((BM, D), lambda i: (i, 0))
    full = pl.BlockSpec((D, D), lambda i: (0, 0))
    bias = pl.BlockSpec((1, D), lambda i: (0, 0))
    out = jax.ShapeDtypeStruct((N_NODES, D), jnp.float32)
    if relu:
        return pl.pallas_call(
            _tc_body1,
            grid=grid,
            in_specs=[row, row, row, deg, deg, full, bias],
            out_specs=[row, row],
            out_shape=[out, out],
        )(p[0], p[1], h, dp[0], dp[1], wt, b)
    return pl.pallas_call(
        _tc_body2,
        grid=grid,
        in_specs=[row, row, row, deg, deg, full, bias],
        out_specs=row,
        out_shape=out,
    )(p[0], p[1], h, dp[0], dp[1], wt, b)


def kernel(feats, edge_index, W1, b1, W2, b2):
    src = edge_index[0].astype(jnp.int32)
    dst = edge_index[1].astype(jnp.int32)
    pad = E_PAD - N_EDGES
    spread = jnp.arange(pad, dtype=jnp.int32) % (NPAD - N_NODES)
    src3 = jnp.concatenate([src, spread]).reshape(NW, K, CH)
    dst3 = jnp.concatenate([dst, N_NODES + spread]).reshape(NW, K, CH)
    w1t = W1.T
    w2t = W2.T
    b1r = b1.reshape(1, D)
    b2r = b2.reshape(1, D)

    dp = _deg_sum(dst3)
    p1 = _seg_sum(src3, dst3, feats)
    h1, r = _tc_layer(p1, dp, feats, w1t, b1r, True)
    p2 = _seg_sum(src3, dst3, r)
    h2 = _tc_layer(p2, dp, r, w2t, b2r, False)
    return (h1, h2)
